# baseline (speedup 1.0000x reference)
import functools

import jax
import jax.numpy as jnp
from jax import lax
from jax.experimental import pallas as pl
from jax.experimental.pallas import tpu as pltpu
from jax.experimental.pallas import tpu_sc as plsc

_NC = 2
_NS = 16
_NW = _NC * _NS
_D = 128
_P = 200
_SUPER = 12800
_PSUPER = _SUPER // 2
_CPS = _PSUPER // _P
_L = 16
_GPB = (_PSUPER // _L) // (_CPS // 2)


@functools.partial(jax.jit, static_argnames=("n",))
def _sc_lookup(idx_flat, pair_table, n):
    per_w = n // _NW
    pairs_w = per_w // 2
    n_super = per_w // _SUPER
    mesh = plsc.VectorSubcoreMesh(core_axis_name="c", subcore_axis_name="s")

    @functools.partial(
        pl.kernel,
        out_type=jax.ShapeDtypeStruct((n // 2, 2, _D), jnp.float32),
        mesh=mesh,
        scratch_types=[
            pltpu.VMEM((_PSUPER,), jnp.int32),
            pltpu.VMEM((_PSUPER,), jnp.int32),
            pltpu.VMEM((_PSUPER,), jnp.int32),
            pltpu.VMEM((_PSUPER,), jnp.int32),
            pltpu.VMEM((_P, 2, _D), jnp.float32),
            pltpu.VMEM((_P, 2, _D), jnp.float32),
            pltpu.VMEM_SHARED((16, 2, _D), jnp.float32),
            pltpu.SemaphoreType.DMA,
            pltpu.SemaphoreType.DMA,
            pltpu.SemaphoreType.DMA,
            pltpu.SemaphoreType.DMA,
        ],
    )
    def k(idx_hbm, ptable_hbm, out_hbm, i0, i1, p0, p1, rows0, rows1,
          table_sh, g0, g1, s0, s1):
        sid = lax.axis_index("s")
        wid = sid * _NC + lax.axis_index("c")
        base = wid * per_w
        pbase = wid * pairs_w
        idxb = (i0, i1)
        pidxb = (p0, p1)
        rows = (rows0, rows1)
        gsem = (g0, g1)
        ssem = (s0, s1)

        @pl.when(sid == 0)
        def _():
            pltpu.sync_copy(ptable_hbm, table_sh)

        plsc.subcore_barrier()

        def gstart(pidx_v, j, b):
            pltpu.async_copy(
                table_sh.at[pidx_v.at[pl.ds(j * _P, _P)]], rows[b], gsem[b])

        def gwait(b):
            pltpu.make_async_copy(
                table_sh.at[p0.at[pl.ds(0, _P)]], rows[b], gsem[b]).wait()

        def sstart(off, b):
            pltpu.async_copy(rows[b], out_hbm.at[pl.ds(off, _P)], ssem[b])

        def swait(b):
            pltpu.make_async_copy(
                rows[b], out_hbm.at[pl.ds(pbase, _P)], ssem[b]).wait()

        def compute_pidx(src_i, dst_p, g_lo, count):
            @pl.loop(0, count)
            def _(gg):
                g = g_lo + gg
                w = src_i[pl.ds(g * _L, _L)]
                dst_p[pl.ds(g * _L, _L)] = 4 * (w & 0xFFFF) + (w >> 16)

        pltpu.sync_copy(idx_hbm.at[pl.ds(pbase, _PSUPER)], idxb[0])
        compute_pidx(idxb[0], pidxb[0], 0, _PSUPER // _L)

        for s in range(n_super):
            cur, nxt = s % 2, (s + 1) % 2
            sbase = pbase + s * _PSUPER
            gstart(pidxb[cur], 0, 0)
            if s + 1 < n_super:
                pltpu.sync_copy(
                    idx_hbm.at[pl.ds(pbase + (s + 1) * _PSUPER, _PSUPER)],
                    idxb[nxt])

            @pl.loop(0, _CPS, step=2)
            def _(j, s=s, cur=cur, nxt=nxt, sbase=sbase):
                @pl.when(j > 0)
                def _():
                    swait(1)
                gstart(pidxb[cur], j + 1, 1)
                if s + 1 < n_super:
                    compute_pidx(idxb[nxt], pidxb[nxt], (j // 2) * _GPB, _GPB)
                gwait(0)
                sstart(sbase + j * _P, 0)
                swait(0)

                @pl.when(j + 2 < _CPS)
                def _():
                    gstart(pidxb[cur], j + 2, 0)
                gwait(1)
                sstart(sbase + (j + 1) * _P, 1)

            swait(1)

    return k(idx_flat, pair_table)


def kernel(inputs, table):
    b, l = inputs.shape
    n = b * l
    table = table.at[2].set(0.0)
    pair_table = jnp.concatenate(
        [jnp.repeat(table, 4, axis=0), jnp.tile(table, (4, 1))],
        axis=1).reshape(16, 2, _D)
    idx_packed = jax.lax.bitcast_convert_type(
        inputs.reshape(n // 2, 2).astype(jnp.int16), jnp.int32)
    out = _sc_lookup(idx_packed, pair_table, n)
    return out.reshape(b, l, _D)


# speedup vs baseline: 1.8714x; 1.8714x over previous
import functools

import jax
import jax.numpy as jnp
from jax import lax
from jax.experimental import pallas as pl
from jax.experimental.pallas import tpu as pltpu
from jax.experimental.pallas import tpu_sc as plsc

_NC = 2
_NS = 16
_NW = _NC * _NS
_D = 128
_P = 200
_SUPER = 6400
_PSUPER = _SUPER // 2
_CPS = _PSUPER // _P
_L = 16
_GPB = (_PSUPER // _L) // (_CPS // 2)


@functools.partial(jax.jit, static_argnames=("n",))
def _sc_lookup(idx_flat, pair_table, n):
    per_w = n // _NW
    pairs_w = per_w // 2
    n_super = per_w // _SUPER
    assert n_super % 2 == 0
    mesh = plsc.VectorSubcoreMesh(core_axis_name="c", subcore_axis_name="s")

    @functools.partial(
        pl.kernel,
        out_type=jax.ShapeDtypeStruct((n // 2, 2, _D), jnp.float32),
        mesh=mesh,
        scratch_types=[
            pltpu.VMEM((_SUPER,), jnp.int32),
            pltpu.VMEM((_SUPER,), jnp.int32),
            pltpu.VMEM((_PSUPER,), jnp.int32),
            pltpu.VMEM((_PSUPER,), jnp.int32),
            pltpu.VMEM((_P, 2, _D), jnp.float32),
            pltpu.VMEM((_P, 2, _D), jnp.float32),
            pltpu.VMEM_SHARED((16, 2, _D), jnp.float32),
            pltpu.SemaphoreType.DMA,
            pltpu.SemaphoreType.DMA,
            pltpu.SemaphoreType.DMA,
            pltpu.SemaphoreType.DMA,
        ],
    )
    def k(idx_hbm, ptable_hbm, out_hbm, i0, i1, p0, p1, rows0, rows1,
          table_sh, g0, g1, s0, s1):
        sid = lax.axis_index("s")
        wid = sid * _NC + lax.axis_index("c")
        base = wid * per_w
        pbase = wid * pairs_w
        idxb = (i0, i1)
        pidxb = (p0, p1)
        rows = (rows0, rows1)
        gsem = (g0, g1)
        ssem = (s0, s1)

        @pl.when(sid == 0)
        def _():
            pltpu.sync_copy(ptable_hbm, table_sh)

        plsc.subcore_barrier()

        def gstart(pidx_v, j, b):
            pltpu.async_copy(
                table_sh.at[pidx_v.at[pl.ds(j * _P, _P)]], rows[b], gsem[b])

        def gwait(b):
            pltpu.make_async_copy(
                table_sh.at[p0.at[pl.ds(0, _P)]], rows[b], gsem[b]).wait()

        def sstart(off, b):
            pltpu.async_copy(rows[b], out_hbm.at[pl.ds(off, _P)], ssem[b])

        def swait(b):
            pltpu.make_async_copy(
                rows[b], out_hbm.at[pl.ds(pbase, _P)], ssem[b]).wait()

        lanes = lax.iota(jnp.int32, _L)
        swp = lanes ^ 1
        evens = (2 * lanes) & (_L - 1)
        lo_half = lanes < (_L // 2)
        _dn = lax.GatherDimensionNumbers(
            offset_dims=(), collapsed_slice_dims=(0,), start_index_map=(0,))

        def xlane(v, pat):
            return lax.gather(v, pat[:, None], _dn, (1,),
                              mode=lax.GatherScatterMode.PROMISE_IN_BOUNDS)

        def compute_pidx(src_i, dst_p, g_lo, count):
            @pl.loop(0, count)
            def _(gg):
                g = g_lo + gg
                va = src_i[pl.ds(g * 2 * _L, _L)]
                vb = src_i[pl.ds(g * 2 * _L + _L, _L)]
                ta = 4 * va + xlane(va, swp)
                tb = 4 * vb + xlane(vb, swp)
                pa = xlane(ta, evens)
                pb = xlane(tb, evens)
                dst_p[pl.ds(g * _L, _L)] = jnp.where(lo_half, pa, pb)

        pltpu.sync_copy(idx_hbm.at[pl.ds(base, _SUPER)], idxb[0])
        compute_pidx(idxb[0], pidxb[0], 0, _PSUPER // _L)

        @pl.loop(0, n_super, step=2)
        def _(s):
            for half in (0, 1):
                sh = s + half
                cur, nxt = half, 1 - half
                sbase = pbase + sh * _PSUPER
                more = sh + 1 < n_super

                gstart(pidxb[cur], 0, 0)
                if half == 0:
                    pltpu.sync_copy(
                        idx_hbm.at[pl.ds(base + (sh + 1) * _SUPER, _SUPER)],
                        idxb[nxt])
                else:
                    @pl.when(more)
                    def _():
                        pltpu.sync_copy(
                            idx_hbm.at[
                                pl.ds(base + (sh + 1) * _SUPER, _SUPER)],
                            idxb[nxt])

                @pl.loop(0, _CPS, step=2)
                def _(j, cur=cur, nxt=nxt, sbase=sbase, half=half, more=more):
                    @pl.when(j > 0)
                    def _():
                        swait(1)
                    gstart(pidxb[cur], j + 1, 1)
                    if half == 0:
                        compute_pidx(idxb[nxt], pidxb[nxt],
                                     (j // 2) * _GPB, _GPB)
                    else:
                        @pl.when(more)
                        def _():
                            compute_pidx(idxb[nxt], pidxb[nxt],
                                         (j // 2) * _GPB, _GPB)
                    gwait(0)
                    sstart(sbase + j * _P, 0)
                    swait(0)

                    @pl.when(j + 2 < _CPS)
                    def _():
                        gstart(pidxb[cur], j + 2, 0)
                    gwait(1)
                    sstart(sbase + (j + 1) * _P, 1)

                swait(1)

    return k(idx_flat, pair_table)


def kernel(inputs, table):
    b, l = inputs.shape
    n = b * l
    table = table.at[2].set(0.0)
    pair_table = jnp.concatenate(
        [jnp.repeat(table, 4, axis=0), jnp.tile(table, (4, 1))],
        axis=1).reshape(16, 2, _D)
    out = _sc_lookup(inputs.reshape(n), pair_table, n)
    return out.reshape(b, l, _D)


# speedup vs baseline: 1.8772x; 1.0031x over previous
import functools

import jax
import jax.numpy as jnp
from jax import lax
from jax.experimental import pallas as pl
from jax.experimental.pallas import tpu as pltpu
from jax.experimental.pallas import tpu_sc as plsc

_NC = 2
_NS = 16
_NW = _NC * _NS
_D = 128
_P = 200
_SUPER = 6400
_PSUPER = _SUPER // 2
_CPS = _PSUPER // _P
_L = 16
_GPB = (_PSUPER // _L) // (_CPS // 2)


@functools.partial(jax.jit, static_argnames=("n",))
def _sc_lookup(idx_flat, pair_table, n):
    per_w = n // _NW
    pairs_w = per_w // 2
    n_super = per_w // _SUPER
    assert n_super % 2 == 0
    mesh = plsc.VectorSubcoreMesh(core_axis_name="c", subcore_axis_name="s")

    @functools.partial(
        pl.kernel,
        out_type=jax.ShapeDtypeStruct((n // 2, 2, _D), jnp.float32),
        mesh=mesh,
        scratch_types=[
            pltpu.VMEM((_SUPER,), jnp.int32),
            pltpu.VMEM((_SUPER,), jnp.int32),
            pltpu.VMEM((_PSUPER,), jnp.int32),
            pltpu.VMEM((_PSUPER,), jnp.int32),
            pltpu.VMEM((_P, 2, _D), jnp.float32),
            pltpu.VMEM((_P, 2, _D), jnp.float32),
            pltpu.VMEM_SHARED((16, 2, _D), jnp.float32),
            pltpu.SemaphoreType.DMA,
            pltpu.SemaphoreType.DMA,
            pltpu.SemaphoreType.DMA,
            pltpu.SemaphoreType.DMA,
        ],
    )
    def k(idx_hbm, ptable_hbm, out_hbm, i0, i1, p0, p1, rows0, rows1,
          table_sh, g0, g1, s0, s1):
        sid = lax.axis_index("s")
        wid = sid * _NC + lax.axis_index("c")
        base = wid * per_w
        pbase = wid * pairs_w
        idxb = (i0, i1)
        pidxb = (p0, p1)
        rows = (rows0, rows1)
        gsem = (g0, g1)
        ssem = (s0, s1)

        @pl.when(sid == 0)
        def _():
            pltpu.sync_copy(ptable_hbm, table_sh)

        plsc.subcore_barrier()

        def gstart(pidx_v, j, b):
            pltpu.async_copy(
                table_sh.at[pidx_v.at[pl.ds(j * _P, _P)]], rows[b], gsem[b])

        def gwait(b):
            pltpu.make_async_copy(
                table_sh.at[p0.at[pl.ds(0, _P)]], rows[b], gsem[b]).wait()

        def sstart(off, b):
            pltpu.async_copy(rows[b], out_hbm.at[pl.ds(off, _P)], ssem[b])

        def swait(b):
            pltpu.make_async_copy(
                rows[b], out_hbm.at[pl.ds(pbase, _P)], ssem[b]).wait()

        lanes = lax.iota(jnp.int32, _L)
        swp = lanes ^ 1
        evens = (2 * lanes) & (_L - 1)
        lo_half = lanes < (_L // 2)
        _dn = lax.GatherDimensionNumbers(
            offset_dims=(), collapsed_slice_dims=(0,), start_index_map=(0,))

        def xlane(v, pat):
            return lax.gather(v, pat[:, None], _dn, (1,),
                              mode=lax.GatherScatterMode.PROMISE_IN_BOUNDS)

        def compute_pidx(src_i, dst_p, g_lo, count):
            @pl.loop(0, count)
            def _(gg):
                g = g_lo + gg
                va = src_i[pl.ds(g * 2 * _L, _L)]
                vb = src_i[pl.ds(g * 2 * _L + _L, _L)]
                ta = 4 * va + xlane(va, swp)
                tb = 4 * vb + xlane(vb, swp)
                pa = xlane(ta, evens)
                pb = xlane(tb, evens)
                dst_p[pl.ds(g * _L, _L)] = jnp.where(lo_half, pa, pb)

        pltpu.sync_copy(idx_hbm.at[pl.ds(base, _SUPER)], idxb[0])
        compute_pidx(idxb[0], pidxb[0], 0, _PSUPER // _L)

        @pl.loop(0, n_super, step=2)
        def _(s):
            for half in (0, 1):
                sh = s + half
                cur, nxt = half, 1 - half
                sbase = pbase + sh * _PSUPER
                more = sh + 1 < n_super

                gstart(pidxb[cur], 0, 0)
                if half == 0:
                    pltpu.sync_copy(
                        idx_hbm.at[pl.ds(base + (sh + 1) * _SUPER, _SUPER)],
                        idxb[nxt])
                else:
                    @pl.when(more)
                    def _():
                        pltpu.sync_copy(
                            idx_hbm.at[
                                pl.ds(base + (sh + 1) * _SUPER, _SUPER)],
                            idxb[nxt])

                _G1 = _GPB // 2
                _G2 = _GPB - _G1

                @pl.loop(0, _CPS, step=2)
                def _(j, cur=cur, nxt=nxt, sbase=sbase, half=half, more=more):
                    @pl.when(j > 0)
                    def _():
                        swait(1)
                    gstart(pidxb[cur], j + 1, 1)
                    gwait(0)
                    sstart(sbase + j * _P, 0)
                    if half == 0:
                        compute_pidx(idxb[nxt], pidxb[nxt],
                                     (j // 2) * _GPB, _G1)
                    else:
                        @pl.when(more)
                        def _():
                            compute_pidx(idxb[nxt], pidxb[nxt],
                                         (j // 2) * _GPB, _G1)
                    swait(0)

                    @pl.when(j + 2 < _CPS)
                    def _():
                        gstart(pidxb[cur], j + 2, 0)
                    gwait(1)
                    sstart(sbase + (j + 1) * _P, 1)
                    if half == 0:
                        compute_pidx(idxb[nxt], pidxb[nxt],
                                     (j // 2) * _GPB + _G1, _G2)
                    else:
                        @pl.when(more)
                        def _():
                            compute_pidx(idxb[nxt], pidxb[nxt],
                                         (j // 2) * _GPB + _G1, _G2)

                swait(1)

    return k(idx_flat, pair_table)


def kernel(inputs, table):
    b, l = inputs.shape
    n = b * l
    table = table.at[2].set(0.0)
    pair_table = jnp.concatenate(
        [jnp.repeat(table, 4, axis=0), jnp.tile(table, (4, 1))],
        axis=1).reshape(16, 2, _D)
    out = _sc_lookup(inputs.reshape(n), pair_table, n)
    return out.reshape(b, l, _D)
